# baseline (speedup 1.0000x reference)
import functools

import jax
import jax.numpy as jnp
from jax import lax
from jax.experimental import pallas as pl
from jax.experimental.pallas import tpu as pltpu
from jax.experimental.pallas import tpu_sc as plsc

_VOCAB = 100000
_EMBED = 128
_SEQ = 200
_BATCH = 4096
_HALF = _SEQ // 2
_LANES = 16
_NB = 4


def _make_kernel():
    info = plsc.get_sparse_core_info()
    nc, ns = info.num_cores, info.num_subcores
    nw = nc * ns
    rows = _BATCH * _SEQ
    per_w = rows // nw
    nseq_w = per_w // _SEQ

    mesh = plsc.VectorSubcoreMesh(core_axis_name="c", subcore_axis_name="s")

    @functools.partial(
        pl.kernel,
        mesh=mesh,
        out_type=jax.ShapeDtypeStruct((rows, _EMBED), jnp.float32),
        scratch_types=[
            pltpu.VMEM((_SEQ, _EMBED), jnp.float32),
            pltpu.VMEM((_NB, 2, _HALF), jnp.int32),
            pltpu.VMEM((_NB, _SEQ, _EMBED), jnp.float32),
        ] + [pltpu.SemaphoreType.DMA] * (3 * _NB),
    )
    def kern(ids_hbm, tok_hbm, pos_hbm, out_hbm, pos_v, idx_v, rows_v,
             *sems):
        sem_i = sems[0:_NB]
        sem_g = sems[_NB:2 * _NB]
        sem_s = sems[2 * _NB:3 * _NB]
        wid = lax.axis_index("s") * nc + lax.axis_index("c")
        row0 = wid * per_w
        irow0 = wid * (per_w // _HALF)

        def idx_dma(g, b):
            return pltpu.make_async_copy(
                ids_hbm.at[pl.ds(irow0 + 2 * g, 2)], idx_v.at[b], sem_i[b])

        def gather_dma(b, h):
            return pltpu.make_async_copy(
                tok_hbm.at[idx_v.at[b, h]],
                rows_v.at[b, pl.ds(h * _HALF, _HALF)], sem_g[b])

        def scatter_dma(g, b):
            return pltpu.make_async_copy(
                rows_v.at[b], out_hbm.at[pl.ds(row0 + g * _SEQ, _SEQ)],
                sem_s[b])

        def gather_start(b):
            gather_dma(b, 0).start()
            gather_dma(b, 1).start()

        def gather_wait(b):
            gather_dma(b, 0).wait()
            gather_dma(b, 1).wait()

        def add_pos(b):
            @plsc.parallel_loop(0, _SEQ, step=1, unroll=8)
            def _rbody(r):
                for c in range(_EMBED // _LANES):
                    sl = pl.ds(c * _LANES, _LANES)
                    plsc.addupdate(rows_v.at[b, r, sl], pos_v[r, sl])

        pltpu.sync_copy(pos_hbm.at[pl.ds(0, _SEQ)], pos_v)

        def step(g, b, w_scat, g_next, i_next):
            if w_scat:
                scatter_dma(0, (b + 1) % _NB).wait()
            if g_next:
                idx_dma(0, (b + 1) % _NB).wait()
                gather_start((b + 1) % _NB)
            if i_next:
                idx_dma(g + 2, (b + 2) % _NB).start()
            gather_wait(b)
            add_pos(b)
            scatter_dma(g, b).start()

        idx_dma(0, 0).start()
        idx_dma(0, 0).wait()
        gather_start(0)
        idx_dma(1, 1).start()

        step(0, 0, False, True, True)
        step(1, 1, False, True, True)
        step(2, 2, False, True, True)
        step(3, 3, True, True, True)

        def loop_body(k, _):
            g = 4 * k + 4
            step(g + 0, 0, True, True, True)
            step(g + 1, 1, True, True, True)
            step(g + 2, 2, True, True, True)
            step(g + 3, 3, True, True, True)
            return _
        lax.fori_loop(0, (nseq_w - 8) // 4, loop_body, 0)

        step(nseq_w - 4, 0, True, True, True)
        step(nseq_w - 3, 1, True, True, True)
        step(nseq_w - 2, 2, True, True, False)
        step(nseq_w - 1, 3, True, False, False)
        scatter_dma(0, 1).wait()
        scatter_dma(0, 2).wait()
        scatter_dma(0, 3).wait()

    return kern


_kern = _make_kernel()


def kernel(input_ids, token_table, position_table):
    rows = _BATCH * _SEQ
    ids = jnp.reshape(input_ids.astype(jnp.int32), (rows // _HALF, _HALF))
    out = _kern(ids, token_table, position_table)
    return jnp.reshape(out, (_BATCH, _SEQ, _EMBED))


# speedup vs baseline: 1.0053x; 1.0053x over previous
import functools

import jax
import jax.numpy as jnp
from jax import lax
from jax.experimental import pallas as pl
from jax.experimental.pallas import tpu as pltpu
from jax.experimental.pallas import tpu_sc as plsc

_VOCAB = 100000
_EMBED = 128
_SEQ = 200
_BATCH = 4096
_HALF = _SEQ // 2
_LANES = 16
_NB = 4


def _make_kernel():
    info = plsc.get_sparse_core_info()
    nc, ns = info.num_cores, info.num_subcores
    nw = nc * ns
    rows = _BATCH * _SEQ
    per_w = rows // nw
    nseq_w = per_w // _SEQ

    mesh = plsc.VectorSubcoreMesh(core_axis_name="c", subcore_axis_name="s")

    @functools.partial(
        pl.kernel,
        mesh=mesh,
        out_type=jax.ShapeDtypeStruct((rows, _EMBED), jnp.float32),
        scratch_types=[
            pltpu.VMEM((_SEQ, _EMBED), jnp.float32),
            pltpu.VMEM((_NB, 2, _HALF), jnp.int32),
            pltpu.VMEM((_NB, _SEQ, _EMBED), jnp.float32),
        ] + [pltpu.SemaphoreType.DMA] * (3 * _NB),
    )
    def kern(ids_hbm, tok_hbm, pos_hbm, out_hbm, pos_v, idx_v, rows_v,
             *sems):
        sem_i = sems[0:_NB]
        sem_g = sems[_NB:2 * _NB]
        sem_s = sems[2 * _NB:3 * _NB]
        wid = lax.axis_index("s") * nc + lax.axis_index("c")
        row0 = wid * per_w
        irow0 = wid * (per_w // _HALF)

        def idx_dma(g, b):
            return pltpu.make_async_copy(
                ids_hbm.at[pl.ds(irow0 + 2 * g, 2)], idx_v.at[b], sem_i[b])

        def gather_dma(b, h):
            return pltpu.make_async_copy(
                tok_hbm.at[idx_v.at[b, h]],
                rows_v.at[b, pl.ds(h * _HALF, _HALF)], sem_g[b])

        def scatter_dma(g, b):
            return pltpu.make_async_copy(
                rows_v.at[b], out_hbm.at[pl.ds(row0 + g * _SEQ, _SEQ)],
                sem_s[b])

        def gather_start(b):
            gather_dma(b, 0).start()
            gather_dma(b, 1).start()

        def gather_wait(b):
            gather_dma(b, 0).wait()
            gather_dma(b, 1).wait()

        def add_pos(b):
            @plsc.parallel_loop(0, _SEQ, step=1, unroll=4)
            def _rbody(r):
                for c in range(_EMBED // _LANES):
                    sl = pl.ds(c * _LANES, _LANES)
                    plsc.addupdate(rows_v.at[b, r, sl], pos_v[r, sl])

        pltpu.sync_copy(pos_hbm.at[pl.ds(0, _SEQ)], pos_v)

        def step(g, b, w_scat, g_next, i_next):
            if w_scat:
                scatter_dma(0, (b + 1) % _NB).wait()
            if g_next:
                idx_dma(0, (b + 1) % _NB).wait()
                gather_start((b + 1) % _NB)
            if i_next:
                idx_dma(g + 2, (b + 2) % _NB).start()
            gather_wait(b)
            add_pos(b)
            scatter_dma(g, b).start()

        idx_dma(0, 0).start()
        idx_dma(0, 0).wait()
        gather_start(0)
        idx_dma(1, 1).start()

        step(0, 0, False, True, True)
        step(1, 1, False, True, True)
        step(2, 2, False, True, True)
        step(3, 3, True, True, True)

        def loop_body(k, _):
            g = 4 * k + 4
            step(g + 0, 0, True, True, True)
            step(g + 1, 1, True, True, True)
            step(g + 2, 2, True, True, True)
            step(g + 3, 3, True, True, True)
            return _
        lax.fori_loop(0, (nseq_w - 8) // 4, loop_body, 0)

        step(nseq_w - 4, 0, True, True, True)
        step(nseq_w - 3, 1, True, True, True)
        step(nseq_w - 2, 2, True, True, False)
        step(nseq_w - 1, 3, True, False, False)
        scatter_dma(0, 1).wait()
        scatter_dma(0, 2).wait()
        scatter_dma(0, 3).wait()

    return kern


_kern = _make_kernel()


def kernel(input_ids, token_table, position_table):
    rows = _BATCH * _SEQ
    ids = jnp.reshape(input_ids.astype(jnp.int32), (rows // _HALF, _HALF))
    out = _kern(ids, token_table, position_table)
    return jnp.reshape(out, (_BATCH, _SEQ, _EMBED))
